# Initial kernel scaffold; baseline (speedup 1.0000x reference)
import jax
import jax.numpy as jnp
from jax.experimental import pallas as pl


def kernel(x, edge_index, edge_attr, batch, glbl_x, pre_n_w, pre_n_b, pre_e_w, pre_e_b, W_stack, att_stack, bias_stack, bn1_g, bn1_b, dgn_lin, dgn_g, dgn_b, ga_w0, ga_b0, ga_w1, ga_b1, ga_w2, ga_b2, post_w, post_b, out_w, out_b):
    raise NotImplementedError("write your pallas kernel here")


# speedup vs baseline: 7.5558x; 7.5558x over previous
import functools
import jax
import jax.numpy as jnp
from jax.experimental import pallas as pl

_N = 10000
_E = 320000
_G = 100
_D = 64
_H = 4
_GC = 5
_GROUPS = 10
_LAMBDA = 0.01
_BM = 2000


def _sp(v):
    return jnp.log(1.0 + jnp.exp(-jnp.abs(v))) + jnp.maximum(v, 0.0)


def _row_spec(bm, ncols):
    return pl.BlockSpec((bm, ncols), lambda i: (i, 0))


def _full_spec(shape):
    nd = len(shape)
    return pl.BlockSpec(shape, lambda i: (0,) * nd)


def _mm_body(x_ref, w_ref, b_ref, o_ref, *, act):
    y = jnp.dot(x_ref[...], w_ref[...], preferred_element_type=jnp.float32)
    y = y + b_ref[...]
    if act:
        y = _sp(y)
    o_ref[...] = y


def _mm(x, w, b, act, bm=_BM):
    m, k = x.shape
    n = w.shape[1]
    grid = m // bm
    return pl.pallas_call(
        functools.partial(_mm_body, act=act),
        grid=(grid,),
        in_specs=[_row_spec(bm, k), _full_spec((k, n)), _full_spec((1, n))],
        out_specs=_row_spec(bm, n),
        out_shape=jax.ShapeDtypeStruct((m, n), jnp.float32),
    )(x, w, b.reshape(1, n))


def _pass1_body(qi_ref, qj_ref, oe_ref, wbot_ref, ai_ref, aj_ref,
                ar_ref, st_ref):
    ew = jnp.dot(oe_ref[...], wbot_ref[...], preferred_element_type=jnp.float32)
    oi = _sp(qi_ref[...] + ew)
    oj = _sp(qj_ref[...] + ew)
    ar = _sp(jnp.dot(oi, ai_ref[...], preferred_element_type=jnp.float32)
             + jnp.dot(oj, aj_ref[...], preferred_element_type=jnp.float32))
    ar_ref[...] = ar
    row = jax.lax.broadcasted_iota(jnp.int32, (8, 8), 0)

    @pl.when(pl.program_id(0) == 0)
    def _():
        st_ref[...] = jnp.where(row == 2, -jnp.inf, 0.0)

    s = jnp.sum(ar, axis=0, keepdims=True)
    ss = jnp.sum(ar * ar, axis=0, keepdims=True)
    mx = jnp.max(ar, axis=0, keepdims=True)
    z = jnp.zeros_like(s)
    ninf = jnp.full_like(s, -jnp.inf)
    addp = jnp.concatenate([s, ss, z, z, z, z, z, z], axis=0)
    maxp = jnp.concatenate([ninf, ninf, mx, ninf, ninf, ninf, ninf, ninf],
                           axis=0)
    prev = st_ref[...]
    st_ref[...] = jnp.where(row == 2, jnp.maximum(prev, maxp), prev + addp)


def _pass2_body(ar_ref, p_ref, e_ref):
    p = p_ref[...]
    mu = p[0:1, :]
    rstd = p[1:2, :]
    g = p[2:3, :]
    b = p[3:4, :]
    mx = p[4:5, :]
    a2 = _sp(g * (ar_ref[...] - mu) * rstd + b)
    e_ref[...] = jnp.exp(a2 - mx)


def _pass3_body(qj_ref, oe_ref, wbot_ref, c_ref, m_ref):
    ew = jnp.dot(oe_ref[...], wbot_ref[...], preferred_element_type=jnp.float32)
    oj = _sp(qj_ref[...] + ew)
    c = c_ref[...]
    acc = oj[:, 0:_D] * c[:, 0:1]
    for h in range(1, _H):
        acc = acc + oj[:, h * _D:(h + 1) * _D] * c[:, h:h + 1]
    m_ref[...] = acc * (1.0 / _H)


def _dgn1_body(ag_ref, bias_ref, lin_ref, o640_ref, hpre_ref, st_ref):
    h = ag_ref[...] + bias_ref[...]
    hpre_ref[...] = h
    logits = jnp.dot(h, lin_ref[...], preferred_element_type=jnp.float32)
    col = jax.lax.broadcasted_iota(jnp.int32, logits.shape, 1)
    logits = jnp.where(col < _GROUPS, logits, -jnp.inf)
    logits = logits - jnp.max(logits, axis=1, keepdims=True)
    ex = jnp.where(col < _GROUPS, jnp.exp(logits), 0.0)
    s = ex / jnp.sum(ex, axis=1, keepdims=True)
    parts = [s[:, g:g + 1] * h for g in range(_GROUPS)]
    o640 = jnp.concatenate(parts, axis=1)
    o640_ref[...] = o640

    @pl.when(pl.program_id(0) == 0)
    def _():
        st_ref[...] = jnp.zeros_like(st_ref)

    sm = jnp.sum(o640, axis=0, keepdims=True)
    sq = jnp.sum(o640 * o640, axis=0, keepdims=True)
    z = jnp.zeros_like(sm)
    st_ref[...] = st_ref[...] + jnp.concatenate(
        [sm, sq, z, z, z, z, z, z], axis=0)


def _dgn2_body(o640_ref, hpre_ref, prev_ref, p_ref, o_ref):
    p = p_ref[...]
    mu = p[0:1, :]
    rstd = p[1:2, :]
    g = p[2:3, :]
    b = p[3:4, :]
    bn = g * (o640_ref[...] - mu) * rstd + b
    acc = bn[:, 0:_D]
    for gi in range(1, _GROUPS):
        acc = acc + bn[:, gi * _D:(gi + 1) * _D]
    o_ref[...] = hpre_ref[...] + _LAMBDA * acc + prev_ref[...]


def _gatt_body(x_ref, gl_ref, w0_ref, b0_ref, w1_ref, b1_ref, w2_ref, b2_ref,
               a_ref, st_ref):
    cat = jnp.concatenate([x_ref[...], gl_ref[...]], axis=1)
    a1 = _sp(jnp.dot(cat, w0_ref[...], preferred_element_type=jnp.float32)
             + b0_ref[...])
    a2 = _sp(jnp.dot(a1, w1_ref[...], preferred_element_type=jnp.float32)
             + b1_ref[...])
    a3 = jnp.dot(a2, w2_ref[...], preferred_element_type=jnp.float32) \
        + b2_ref[...]
    a_ref[...] = a3
    row = jax.lax.broadcasted_iota(jnp.int32, (8, 8), 0)

    @pl.when(pl.program_id(0) == 0)
    def _():
        st_ref[...] = jnp.full_like(st_ref, -jnp.inf)

    mx = jnp.max(a3, axis=0, keepdims=True)
    ninf = jnp.full_like(mx, -jnp.inf)
    maxp = jnp.concatenate([mx] + [ninf] * 7, axis=0)
    st_ref[...] = jnp.maximum(st_ref[...], maxp)


def kernel(x, edge_index, edge_attr, batch, glbl_x, pre_n_w, pre_n_b,
           pre_e_w, pre_e_b, W_stack, att_stack, bias_stack, bn1_g, bn1_b,
           dgn_lin, dgn_g, dgn_b, ga_w0, ga_b0, ga_w1, ga_b1, ga_w2, ga_b2,
           post_w, post_b, out_w, out_b):
    f32 = jnp.float32
    idx_i = edge_index[0].astype(jnp.int32)
    idx_j = edge_index[1].astype(jnp.int32)
    batch = batch.astype(jnp.int32)

    out_x = _mm(x, pre_n_w, pre_n_b, True)
    out_e = _mm(edge_attr, pre_e_w, pre_e_b, True)

    prev = out_x
    zeros256 = jnp.zeros((256,), f32)
    for l in range(_GC):
        W = W_stack[l]
        w_top = W[:_D]
        w_bot = W[_D:]
        att = att_stack[l][0]
        ai = jnp.zeros((_H * _D, 8), f32)
        aj = jnp.zeros((_H * _D, 8), f32)
        for h in range(_H):
            ai = ai.at[h * _D:(h + 1) * _D, h].set(att[h, :_D])
            aj = aj.at[h * _D:(h + 1) * _D, h].set(att[h, _D:])

        q = _mm(out_x, w_top, zeros256, False)
        qi = jnp.take(q, idx_i, axis=0)
        qj = jnp.take(q, idx_j, axis=0)

        grid_e = _E // _BM
        ar, st = pl.pallas_call(
            _pass1_body,
            grid=(grid_e,),
            in_specs=[_row_spec(_BM, 256), _row_spec(_BM, 256),
                      _row_spec(_BM, _D), _full_spec((_D, 256)),
                      _full_spec((256, 8)), _full_spec((256, 8))],
            out_specs=[_row_spec(_BM, 8), _full_spec((8, 8))],
            out_shape=[jax.ShapeDtypeStruct((_E, 8), f32),
                       jax.ShapeDtypeStruct((8, 8), f32)],
        )(qi, qj, out_e, w_bot, ai, aj)

        mu = st[0] / _E
        var = st[1] / _E - mu * mu
        rstd = 1.0 / jnp.sqrt(var + 1e-5)
        g8 = jnp.pad(bn1_g[l], (0, 4))
        b8 = jnp.pad(bn1_b[l], (0, 4))
        mx2 = _sp(g8 * (st[2] - mu) * rstd + b8)
        p = jnp.stack([mu, rstd, g8, b8, mx2,
                       jnp.zeros_like(mu), jnp.zeros_like(mu),
                       jnp.zeros_like(mu)], axis=0)

        e = pl.pallas_call(
            _pass2_body,
            grid=(grid_e,),
            in_specs=[_row_spec(_BM, 8), _full_spec((8, 8))],
            out_specs=_row_spec(_BM, 8),
            out_shape=jax.ShapeDtypeStruct((_E, 8), f32),
        )(ar, p)

        e4 = e[:, :_H]
        s = jax.ops.segment_sum(e4, idx_i, num_segments=_N)
        coef = e4 / (jnp.take(s, idx_i, axis=0) + 1e-16)
        coef8 = jnp.pad(coef, ((0, 0), (0, 4)))

        msg = pl.pallas_call(
            _pass3_body,
            grid=(grid_e,),
            in_specs=[_row_spec(_BM, 256), _row_spec(_BM, _D),
                      _full_spec((_D, 256)), _row_spec(_BM, 8)],
            out_specs=_row_spec(_BM, _D),
            out_shape=jax.ShapeDtypeStruct((_E, _D), f32),
        )(qj, out_e, w_bot, coef8)

        aggr = jax.ops.segment_sum(msg, idx_i, num_segments=_N)

        linp = jnp.pad(dgn_lin[l], ((0, 0), (0, 16 - _GROUPS)))
        grid_n = _N // _BM
        o640, hpre, st2 = pl.pallas_call(
            _dgn1_body,
            grid=(grid_n,),
            in_specs=[_row_spec(_BM, _D), _full_spec((1, _D)),
                      _full_spec((_D, 16))],
            out_specs=[_row_spec(_BM, _GROUPS * _D), _row_spec(_BM, _D),
                       _full_spec((8, _GROUPS * _D))],
            out_shape=[jax.ShapeDtypeStruct((_N, _GROUPS * _D), f32),
                       jax.ShapeDtypeStruct((_N, _D), f32),
                       jax.ShapeDtypeStruct((8, _GROUPS * _D), f32)],
        )(aggr, bias_stack[l].reshape(1, _D), linp)

        mu2 = st2[0] / _N
        var2 = st2[1] / _N - mu2 * mu2
        rstd2 = 1.0 / jnp.sqrt(var2 + 1e-5)
        p2 = jnp.stack([mu2, rstd2, dgn_g[l], dgn_b[l],
                        jnp.zeros_like(mu2), jnp.zeros_like(mu2),
                        jnp.zeros_like(mu2), jnp.zeros_like(mu2)], axis=0)

        out_x = pl.pallas_call(
            _dgn2_body,
            grid=(grid_n,),
            in_specs=[_row_spec(_BM, _GROUPS * _D), _row_spec(_BM, _D),
                      _row_spec(_BM, _D), _full_spec((8, _GROUPS * _D))],
            out_specs=_row_spec(_BM, _D),
            out_shape=jax.ShapeDtypeStruct((_N, _D), f32),
        )(o640, hpre, prev, p2)
        prev = out_x

    glp = jnp.pad(glbl_x, ((0, 0), (0, 112 - glbl_x.shape[1])))
    w0p = jnp.pad(ga_w0, ((0, 176 - ga_w0.shape[0]), (0, 0)))
    w2p = jnp.pad(ga_w2, ((0, 0), (0, 7)))
    b2p = jnp.pad(ga_b2, (0, 7)).reshape(1, 8)
    grid_n = _N // _BM
    a3, stf = pl.pallas_call(
        _gatt_body,
        grid=(grid_n,),
        in_specs=[_row_spec(_BM, _D), _row_spec(_BM, 112),
                  _full_spec((176, _D)), _full_spec((1, _D)),
                  _full_spec((_D, _D)), _full_spec((1, _D)),
                  _full_spec((_D, 8)), _full_spec((1, 8))],
        out_specs=[_row_spec(_BM, 8), _full_spec((8, 8))],
        out_shape=[jax.ShapeDtypeStruct((_N, 8), f32),
                   jax.ShapeDtypeStruct((8, 8), f32)],
    )(out_x, glp, w0p, ga_b0.reshape(1, _D), ga_w1, ga_b1.reshape(1, _D),
      w2p, b2p)

    mg = stf[0, 0]
    ea = jnp.exp(a3[:, 0] - mg)
    sg = jax.ops.segment_sum(ea, batch, num_segments=_G)
    coefg = (ea / (jnp.take(sg, batch) + 1e-16))[:, None]
    pooled = jax.ops.segment_sum(out_x * coefg, batch, num_segments=_G)

    pooled = jnp.pad(pooled, ((0, 4), (0, 0)))
    hf = _mm(pooled, post_w, post_b, True, bm=104)
    out_wp = jnp.pad(out_w, ((0, 0), (0, 7)))
    out_bp = jnp.pad(out_b, (0, 7))
    res = _mm(hf, out_wp, out_bp, False, bm=104)
    return res[:_G, 0]


# speedup vs baseline: 10.4168x; 1.3787x over previous
import functools
import jax
import jax.numpy as jnp
from jax.experimental import pallas as pl
from jax.experimental.pallas import tpu as pltpu
from jax.experimental.pallas import tpu_sc as plsc

_N = 10000
_E = 320000
_G = 100
_D = 64
_H = 4
_GC = 5
_GROUPS = 10
_LAMBDA = 0.01
_BM = 2000


def _sp(v):
    return jnp.log(1.0 + jnp.exp(-jnp.abs(v))) + jnp.maximum(v, 0.0)


def _row_spec(bm, ncols):
    return pl.BlockSpec((bm, ncols), lambda i: (i, 0))


def _full_spec(shape):
    nd = len(shape)
    return pl.BlockSpec(shape, lambda i: (0,) * nd)


def _mm_body(x_ref, w_ref, b_ref, o_ref, *, act):
    y = jnp.dot(x_ref[...], w_ref[...], preferred_element_type=jnp.float32)
    y = y + b_ref[...]
    if act:
        y = _sp(y)
    o_ref[...] = y


def _mm(x, w, b, act, bm=_BM):
    m, k = x.shape
    n = w.shape[1]
    grid = m // bm
    return pl.pallas_call(
        functools.partial(_mm_body, act=act),
        grid=(grid,),
        in_specs=[_row_spec(bm, k), _full_spec((k, n)), _full_spec((1, n))],
        out_specs=_row_spec(bm, n),
        out_shape=jax.ShapeDtypeStruct((m, n), jnp.float32),
    )(x, w, b.reshape(1, n))


def _pass1_body(qi_ref, qj_ref, oe_ref, wbot_ref, ai_ref, aj_ref,
                ar_ref, st_ref):
    ew = jnp.dot(oe_ref[...], wbot_ref[...], preferred_element_type=jnp.float32)
    oi = _sp(qi_ref[...] + ew)
    oj = _sp(qj_ref[...] + ew)
    ar = _sp(jnp.dot(oi, ai_ref[...], preferred_element_type=jnp.float32)
             + jnp.dot(oj, aj_ref[...], preferred_element_type=jnp.float32))
    ar_ref[...] = ar
    row = jax.lax.broadcasted_iota(jnp.int32, (8, 8), 0)

    @pl.when(pl.program_id(0) == 0)
    def _():
        st_ref[...] = jnp.where(row == 2, -jnp.inf, 0.0)

    s = jnp.sum(ar, axis=0, keepdims=True)
    ss = jnp.sum(ar * ar, axis=0, keepdims=True)
    mx = jnp.max(ar, axis=0, keepdims=True)
    z = jnp.zeros_like(s)
    ninf = jnp.full_like(s, -jnp.inf)
    addp = jnp.concatenate([s, ss, z, z, z, z, z, z], axis=0)
    maxp = jnp.concatenate([ninf, ninf, mx, ninf, ninf, ninf, ninf, ninf],
                           axis=0)
    prev = st_ref[...]
    st_ref[...] = jnp.where(row == 2, jnp.maximum(prev, maxp), prev + addp)


def _pass2_body(ar_ref, p_ref, e_ref):
    p = p_ref[...]
    mu = p[0:1, :]
    rstd = p[1:2, :]
    g = p[2:3, :]
    b = p[3:4, :]
    mx = p[4:5, :]
    a2 = _sp(g * (ar_ref[...] - mu) * rstd + b)
    e_ref[...] = jnp.exp(a2 - mx)


def _pass3_body(qj_ref, oe_ref, wbot_ref, c_ref, m_ref):
    ew = jnp.dot(oe_ref[...], wbot_ref[...], preferred_element_type=jnp.float32)
    oj = _sp(qj_ref[...] + ew)
    c = c_ref[...]
    acc = oj[:, 0:_D] * c[:, 0:1]
    for h in range(1, _H):
        acc = acc + oj[:, h * _D:(h + 1) * _D] * c[:, h:h + 1]
    m_ref[...] = acc * (1.0 / _H)


def _dgn1_body(ag_ref, bias_ref, lin_ref, o640_ref, hpre_ref, st_ref):
    h = ag_ref[...] + bias_ref[...]
    hpre_ref[...] = h
    logits = jnp.dot(h, lin_ref[...], preferred_element_type=jnp.float32)
    col = jax.lax.broadcasted_iota(jnp.int32, logits.shape, 1)
    logits = jnp.where(col < _GROUPS, logits, -jnp.inf)
    logits = logits - jnp.max(logits, axis=1, keepdims=True)
    ex = jnp.where(col < _GROUPS, jnp.exp(logits), 0.0)
    s = ex / jnp.sum(ex, axis=1, keepdims=True)
    parts = [s[:, g:g + 1] * h for g in range(_GROUPS)]
    o640 = jnp.concatenate(parts, axis=1)
    o640_ref[...] = o640

    @pl.when(pl.program_id(0) == 0)
    def _():
        st_ref[...] = jnp.zeros_like(st_ref)

    sm = jnp.sum(o640, axis=0, keepdims=True)
    sq = jnp.sum(o640 * o640, axis=0, keepdims=True)
    z = jnp.zeros_like(sm)
    st_ref[...] = st_ref[...] + jnp.concatenate(
        [sm, sq, z, z, z, z, z, z], axis=0)


def _dgn2_body(o640_ref, hpre_ref, prev_ref, p_ref, o_ref):
    p = p_ref[...]
    mu = p[0:1, :]
    rstd = p[1:2, :]
    g = p[2:3, :]
    b = p[3:4, :]
    bn = g * (o640_ref[...] - mu) * rstd + b
    acc = bn[:, 0:_D]
    for gi in range(1, _GROUPS):
        acc = acc + bn[:, gi * _D:(gi + 1) * _D]
    o_ref[...] = hpre_ref[...] + _LAMBDA * acc + prev_ref[...]


def _gatt_body(x_ref, gl_ref, w0_ref, b0_ref, w1_ref, b1_ref, w2_ref, b2_ref,
               a_ref, st_ref):
    cat = jnp.concatenate([x_ref[...], gl_ref[...]], axis=1)
    a1 = _sp(jnp.dot(cat, w0_ref[...], preferred_element_type=jnp.float32)
             + b0_ref[...])
    a2 = _sp(jnp.dot(a1, w1_ref[...], preferred_element_type=jnp.float32)
             + b1_ref[...])
    a3 = jnp.dot(a2, w2_ref[...], preferred_element_type=jnp.float32) \
        + b2_ref[...]
    a_ref[...] = a3
    row = jax.lax.broadcasted_iota(jnp.int32, (8, 8), 0)

    @pl.when(pl.program_id(0) == 0)
    def _():
        st_ref[...] = jnp.full_like(st_ref, -jnp.inf)

    mx = jnp.max(a3, axis=0, keepdims=True)
    ninf = jnp.full_like(mx, -jnp.inf)
    maxp = jnp.concatenate([mx] + [ninf] * 7, axis=0)
    st_ref[...] = jnp.maximum(st_ref[...], maxp)


def _sc_gather(table, idx):
    info = plsc.get_sparse_core_info()
    nw = info.num_cores * info.num_subcores
    e, d = idx.shape[0], table.shape[1]
    b_per_w = e // nw
    ch = 200
    nch = b_per_w // ch
    mesh = plsc.VectorSubcoreMesh(core_axis_name="c", subcore_axis_name="s")

    @functools.partial(
        pl.kernel, mesh=mesh,
        out_type=jax.ShapeDtypeStruct((e, d), jnp.float32),
        scratch_types=[pltpu.VMEM((ch,), jnp.int32),
                       pltpu.VMEM((ch, d), jnp.float32),
                       pltpu.SemaphoreType.DMA],
    )
    def k(table_hbm, idx_hbm, out_hbm, idx_v, rows_v, sem):
        wid = jax.lax.axis_index("s") * info.num_cores + jax.lax.axis_index("c")
        base = wid * b_per_w

        def body(i, carry):
            off = base + i * ch
            pltpu.sync_copy(idx_hbm.at[pl.ds(off, ch)], idx_v)
            pltpu.async_copy(table_hbm.at[idx_v], rows_v, sem).wait()
            pltpu.sync_copy(rows_v, out_hbm.at[pl.ds(off, ch)])
            return carry

        jax.lax.fori_loop(0, nch, body, 0)

    return k(table, idx)


def kernel(x, edge_index, edge_attr, batch, glbl_x, pre_n_w, pre_n_b,
           pre_e_w, pre_e_b, W_stack, att_stack, bias_stack, bn1_g, bn1_b,
           dgn_lin, dgn_g, dgn_b, ga_w0, ga_b0, ga_w1, ga_b1, ga_w2, ga_b2,
           post_w, post_b, out_w, out_b):
    f32 = jnp.float32
    idx_i = edge_index[0].astype(jnp.int32)
    idx_j = edge_index[1].astype(jnp.int32)
    batch = batch.astype(jnp.int32)

    out_x = _mm(x, pre_n_w, pre_n_b, True)
    out_e = _mm(edge_attr, pre_e_w, pre_e_b, True)

    prev = out_x
    zeros256 = jnp.zeros((256,), f32)
    for l in range(_GC):
        W = W_stack[l]
        w_top = W[:_D]
        w_bot = W[_D:]
        att = att_stack[l][0]
        ai = jnp.zeros((_H * _D, 8), f32)
        aj = jnp.zeros((_H * _D, 8), f32)
        for h in range(_H):
            ai = ai.at[h * _D:(h + 1) * _D, h].set(att[h, :_D])
            aj = aj.at[h * _D:(h + 1) * _D, h].set(att[h, _D:])

        q = _mm(out_x, w_top, zeros256, False)
        qi = _sc_gather(q, idx_i)
        qj = _sc_gather(q, idx_j)

        grid_e = _E // _BM
        ar, st = pl.pallas_call(
            _pass1_body,
            grid=(grid_e,),
            in_specs=[_row_spec(_BM, 256), _row_spec(_BM, 256),
                      _row_spec(_BM, _D), _full_spec((_D, 256)),
                      _full_spec((256, 8)), _full_spec((256, 8))],
            out_specs=[_row_spec(_BM, 8), _full_spec((8, 8))],
            out_shape=[jax.ShapeDtypeStruct((_E, 8), f32),
                       jax.ShapeDtypeStruct((8, 8), f32)],
        )(qi, qj, out_e, w_bot, ai, aj)

        mu = st[0] / _E
        var = st[1] / _E - mu * mu
        rstd = 1.0 / jnp.sqrt(var + 1e-5)
        g8 = jnp.pad(bn1_g[l], (0, 4))
        b8 = jnp.pad(bn1_b[l], (0, 4))
        mx2 = _sp(g8 * (st[2] - mu) * rstd + b8)
        p = jnp.stack([mu, rstd, g8, b8, mx2,
                       jnp.zeros_like(mu), jnp.zeros_like(mu),
                       jnp.zeros_like(mu)], axis=0)

        e = pl.pallas_call(
            _pass2_body,
            grid=(grid_e,),
            in_specs=[_row_spec(_BM, 8), _full_spec((8, 8))],
            out_specs=_row_spec(_BM, 8),
            out_shape=jax.ShapeDtypeStruct((_E, 8), f32),
        )(ar, p)

        e4 = e[:, :_H]
        s = jax.ops.segment_sum(e4, idx_i, num_segments=_N)
        coef = e4 / (jnp.take(s, idx_i, axis=0) + 1e-16)
        coef8 = jnp.pad(coef, ((0, 0), (0, 4)))

        msg = pl.pallas_call(
            _pass3_body,
            grid=(grid_e,),
            in_specs=[_row_spec(_BM, 256), _row_spec(_BM, _D),
                      _full_spec((_D, 256)), _row_spec(_BM, 8)],
            out_specs=_row_spec(_BM, _D),
            out_shape=jax.ShapeDtypeStruct((_E, _D), f32),
        )(qj, out_e, w_bot, coef8)

        aggr = jax.ops.segment_sum(msg, idx_i, num_segments=_N)

        linp = jnp.pad(dgn_lin[l], ((0, 0), (0, 16 - _GROUPS)))
        grid_n = _N // _BM
        o640, hpre, st2 = pl.pallas_call(
            _dgn1_body,
            grid=(grid_n,),
            in_specs=[_row_spec(_BM, _D), _full_spec((1, _D)),
                      _full_spec((_D, 16))],
            out_specs=[_row_spec(_BM, _GROUPS * _D), _row_spec(_BM, _D),
                       _full_spec((8, _GROUPS * _D))],
            out_shape=[jax.ShapeDtypeStruct((_N, _GROUPS * _D), f32),
                       jax.ShapeDtypeStruct((_N, _D), f32),
                       jax.ShapeDtypeStruct((8, _GROUPS * _D), f32)],
        )(aggr, bias_stack[l].reshape(1, _D), linp)

        mu2 = st2[0] / _N
        var2 = st2[1] / _N - mu2 * mu2
        rstd2 = 1.0 / jnp.sqrt(var2 + 1e-5)
        p2 = jnp.stack([mu2, rstd2, dgn_g[l], dgn_b[l],
                        jnp.zeros_like(mu2), jnp.zeros_like(mu2),
                        jnp.zeros_like(mu2), jnp.zeros_like(mu2)], axis=0)

        out_x = pl.pallas_call(
            _dgn2_body,
            grid=(grid_n,),
            in_specs=[_row_spec(_BM, _GROUPS * _D), _row_spec(_BM, _D),
                      _row_spec(_BM, _D), _full_spec((8, _GROUPS * _D))],
            out_specs=_row_spec(_BM, _D),
            out_shape=jax.ShapeDtypeStruct((_N, _D), f32),
        )(o640, hpre, prev, p2)
        prev = out_x

    glp = jnp.pad(glbl_x, ((0, 0), (0, 112 - glbl_x.shape[1])))
    w0p = jnp.pad(ga_w0, ((0, 176 - ga_w0.shape[0]), (0, 0)))
    w2p = jnp.pad(ga_w2, ((0, 0), (0, 7)))
    b2p = jnp.pad(ga_b2, (0, 7)).reshape(1, 8)
    grid_n = _N // _BM
    a3, stf = pl.pallas_call(
        _gatt_body,
        grid=(grid_n,),
        in_specs=[_row_spec(_BM, _D), _row_spec(_BM, 112),
                  _full_spec((176, _D)), _full_spec((1, _D)),
                  _full_spec((_D, _D)), _full_spec((1, _D)),
                  _full_spec((_D, 8)), _full_spec((1, 8))],
        out_specs=[_row_spec(_BM, 8), _full_spec((8, 8))],
        out_shape=[jax.ShapeDtypeStruct((_N, 8), f32),
                   jax.ShapeDtypeStruct((8, 8), f32)],
    )(out_x, glp, w0p, ga_b0.reshape(1, _D), ga_w1, ga_b1.reshape(1, _D),
      w2p, b2p)

    mg = stf[0, 0]
    ea = jnp.exp(a3[:, 0] - mg)
    sg = jax.ops.segment_sum(ea, batch, num_segments=_G)
    coefg = (ea / (jnp.take(sg, batch) + 1e-16))[:, None]
    pooled = jax.ops.segment_sum(out_x * coefg, batch, num_segments=_G)

    pooled = jnp.pad(pooled, ((0, 4), (0, 0)))
    hf = _mm(pooled, post_w, post_b, True, bm=104)
    out_wp = jnp.pad(out_w, ((0, 0), (0, 7)))
    out_bp = jnp.pad(out_b, (0, 7))
    res = _mm(hf, out_wp, out_bp, False, bm=104)
    return res[:_G, 0]


# speedup vs baseline: 10.5922x; 1.0168x over previous
import functools
import jax
import jax.numpy as jnp
from jax.experimental import pallas as pl
from jax.experimental.pallas import tpu as pltpu
from jax.experimental.pallas import tpu_sc as plsc

_N = 10000
_E = 320000
_G = 100
_D = 64
_H = 4
_GC = 5
_GROUPS = 10
_LAMBDA = 0.01
_BM = 2000


def _sp(v):
    return jnp.log(1.0 + jnp.exp(-jnp.abs(v))) + jnp.maximum(v, 0.0)


def _row_spec(bm, ncols):
    return pl.BlockSpec((bm, ncols), lambda i: (i, 0))


def _full_spec(shape):
    nd = len(shape)
    return pl.BlockSpec(shape, lambda i: (0,) * nd)


def _mm_body(x_ref, w_ref, b_ref, o_ref, *, act):
    y = jnp.dot(x_ref[...], w_ref[...], preferred_element_type=jnp.float32)
    y = y + b_ref[...]
    if act:
        y = _sp(y)
    o_ref[...] = y


def _mm(x, w, b, act, bm=_BM):
    m, k = x.shape
    n = w.shape[1]
    grid = m // bm
    return pl.pallas_call(
        functools.partial(_mm_body, act=act),
        grid=(grid,),
        in_specs=[_row_spec(bm, k), _full_spec((k, n)), _full_spec((1, n))],
        out_specs=_row_spec(bm, n),
        out_shape=jax.ShapeDtypeStruct((m, n), jnp.float32),
    )(x, w, b.reshape(1, n))


def _pass1_body(qi_ref, qj_ref, oe_ref, wbot_ref, ai_ref, aj_ref,
                ar_ref, st_ref):
    ew = jnp.dot(oe_ref[...], wbot_ref[...], preferred_element_type=jnp.float32)
    oi = _sp(qi_ref[...] + ew)
    oj = _sp(qj_ref[...] + ew)
    ar = _sp(jnp.dot(oi, ai_ref[...], preferred_element_type=jnp.float32)
             + jnp.dot(oj, aj_ref[...], preferred_element_type=jnp.float32))
    ar_ref[...] = ar
    row = jax.lax.broadcasted_iota(jnp.int32, (8, 8), 0)

    @pl.when(pl.program_id(0) == 0)
    def _():
        st_ref[...] = jnp.where(row == 2, -jnp.inf, 0.0)

    s = jnp.sum(ar, axis=0, keepdims=True)
    ss = jnp.sum(ar * ar, axis=0, keepdims=True)
    mx = jnp.max(ar, axis=0, keepdims=True)
    z = jnp.zeros_like(s)
    ninf = jnp.full_like(s, -jnp.inf)
    addp = jnp.concatenate([s, ss, z, z, z, z, z, z], axis=0)
    maxp = jnp.concatenate([ninf, ninf, mx, ninf, ninf, ninf, ninf, ninf],
                           axis=0)
    prev = st_ref[...]
    st_ref[...] = jnp.where(row == 2, jnp.maximum(prev, maxp), prev + addp)


def _pass2_body(ar_ref, p_ref, e_ref):
    p = p_ref[...]
    mu = p[0:1, :]
    rstd = p[1:2, :]
    g = p[2:3, :]
    b = p[3:4, :]
    mx = p[4:5, :]
    a2 = _sp(g * (ar_ref[...] - mu) * rstd + b)
    e_ref[...] = jnp.exp(a2 - mx)


def _pass3_body(qj_ref, oe_ref, wbot_ref, c_ref, m_ref):
    ew = jnp.dot(oe_ref[...], wbot_ref[...], preferred_element_type=jnp.float32)
    oj = _sp(qj_ref[...] + ew)
    c = c_ref[...]
    acc = oj[:, 0:_D] * c[:, 0:1]
    for h in range(1, _H):
        acc = acc + oj[:, h * _D:(h + 1) * _D] * c[:, h:h + 1]
    m_ref[...] = acc * (1.0 / _H)


def _dgn1_body(ag_ref, bias_ref, lin_ref, o640_ref, hpre_ref, st_ref):
    h = ag_ref[...] + bias_ref[...]
    hpre_ref[...] = h
    logits = jnp.dot(h, lin_ref[...], preferred_element_type=jnp.float32)
    col = jax.lax.broadcasted_iota(jnp.int32, logits.shape, 1)
    logits = jnp.where(col < _GROUPS, logits, -jnp.inf)
    logits = logits - jnp.max(logits, axis=1, keepdims=True)
    ex = jnp.where(col < _GROUPS, jnp.exp(logits), 0.0)
    s = ex / jnp.sum(ex, axis=1, keepdims=True)
    parts = [s[:, g:g + 1] * h for g in range(_GROUPS)]
    o640 = jnp.concatenate(parts, axis=1)
    o640_ref[...] = o640

    @pl.when(pl.program_id(0) == 0)
    def _():
        st_ref[...] = jnp.zeros_like(st_ref)

    sm = jnp.sum(o640, axis=0, keepdims=True)
    sq = jnp.sum(o640 * o640, axis=0, keepdims=True)
    z = jnp.zeros_like(sm)
    st_ref[...] = st_ref[...] + jnp.concatenate(
        [sm, sq, z, z, z, z, z, z], axis=0)


def _dgn2_body(o640_ref, hpre_ref, prev_ref, p_ref, o_ref):
    p = p_ref[...]
    mu = p[0:1, :]
    rstd = p[1:2, :]
    g = p[2:3, :]
    b = p[3:4, :]
    bn = g * (o640_ref[...] - mu) * rstd + b
    acc = bn[:, 0:_D]
    for gi in range(1, _GROUPS):
        acc = acc + bn[:, gi * _D:(gi + 1) * _D]
    o_ref[...] = hpre_ref[...] + _LAMBDA * acc + prev_ref[...]


def _gatt_body(x_ref, gl_ref, w0_ref, b0_ref, w1_ref, b1_ref, w2_ref, b2_ref,
               a_ref, st_ref):
    cat = jnp.concatenate([x_ref[...], gl_ref[...]], axis=1)
    a1 = _sp(jnp.dot(cat, w0_ref[...], preferred_element_type=jnp.float32)
             + b0_ref[...])
    a2 = _sp(jnp.dot(a1, w1_ref[...], preferred_element_type=jnp.float32)
             + b1_ref[...])
    a3 = jnp.dot(a2, w2_ref[...], preferred_element_type=jnp.float32) \
        + b2_ref[...]
    a_ref[...] = a3
    row = jax.lax.broadcasted_iota(jnp.int32, (8, 8), 0)

    @pl.when(pl.program_id(0) == 0)
    def _():
        st_ref[...] = jnp.full_like(st_ref, -jnp.inf)

    mx = jnp.max(a3, axis=0, keepdims=True)
    ninf = jnp.full_like(mx, -jnp.inf)
    maxp = jnp.concatenate([mx] + [ninf] * 7, axis=0)
    st_ref[...] = jnp.maximum(st_ref[...], maxp)


def _sc_gather(table, idx):
    info = plsc.get_sparse_core_info()
    nw = info.num_cores * info.num_subcores
    e, d = idx.shape[0], table.shape[1]
    b_per_w = e // nw
    ch = 400
    nch = b_per_w // ch
    mesh = plsc.VectorSubcoreMesh(core_axis_name="c", subcore_axis_name="s")

    @functools.partial(
        pl.kernel, mesh=mesh,
        out_type=jax.ShapeDtypeStruct((e, d), jnp.float32),
        scratch_types=[pltpu.VMEM((ch,), jnp.int32),
                       pltpu.VMEM((ch, d), jnp.float32),
                       pltpu.SemaphoreType.DMA],
    )
    def k(table_hbm, idx_hbm, out_hbm, idx_v, rows_v, sem):
        wid = jax.lax.axis_index("s") * info.num_cores + jax.lax.axis_index("c")
        base = wid * b_per_w

        def body(i, carry):
            off = base + i * ch
            pltpu.sync_copy(idx_hbm.at[pl.ds(off, ch)], idx_v)
            pltpu.async_copy(table_hbm.at[idx_v], rows_v, sem).wait()
            pltpu.sync_copy(rows_v, out_hbm.at[pl.ds(off, ch)])
            return carry

        jax.lax.fori_loop(0, nch, body, 0)

    return k(table, idx)


def kernel(x, edge_index, edge_attr, batch, glbl_x, pre_n_w, pre_n_b,
           pre_e_w, pre_e_b, W_stack, att_stack, bias_stack, bn1_g, bn1_b,
           dgn_lin, dgn_g, dgn_b, ga_w0, ga_b0, ga_w1, ga_b1, ga_w2, ga_b2,
           post_w, post_b, out_w, out_b):
    f32 = jnp.float32
    idx_i = edge_index[0].astype(jnp.int32)
    idx_j = edge_index[1].astype(jnp.int32)
    batch = batch.astype(jnp.int32)

    out_x = _mm(x, pre_n_w, pre_n_b, True)
    out_e = _mm(edge_attr, pre_e_w, pre_e_b, True)
    idx_cat = jnp.concatenate([idx_i, idx_j])

    prev = out_x
    zeros256 = jnp.zeros((256,), f32)
    for l in range(_GC):
        W = W_stack[l]
        w_top = W[:_D]
        w_bot = W[_D:]
        att = att_stack[l][0]
        ai = jnp.zeros((_H * _D, 8), f32)
        aj = jnp.zeros((_H * _D, 8), f32)
        for h in range(_H):
            ai = ai.at[h * _D:(h + 1) * _D, h].set(att[h, :_D])
            aj = aj.at[h * _D:(h + 1) * _D, h].set(att[h, _D:])

        q = _mm(out_x, w_top, zeros256, False)
        qcat = _sc_gather(q, idx_cat)

        grid_e = _E // _BM
        qj_spec = pl.BlockSpec((_BM, 256), lambda i: (i + _E // _BM, 0))
        ar, st = pl.pallas_call(
            _pass1_body,
            grid=(grid_e,),
            in_specs=[_row_spec(_BM, 256), qj_spec,
                      _row_spec(_BM, _D), _full_spec((_D, 256)),
                      _full_spec((256, 8)), _full_spec((256, 8))],
            out_specs=[_row_spec(_BM, 8), _full_spec((8, 8))],
            out_shape=[jax.ShapeDtypeStruct((_E, 8), f32),
                       jax.ShapeDtypeStruct((8, 8), f32)],
        )(qcat, qcat, out_e, w_bot, ai, aj)

        mu = st[0] / _E
        var = st[1] / _E - mu * mu
        rstd = 1.0 / jnp.sqrt(var + 1e-5)
        g8 = jnp.pad(bn1_g[l], (0, 4))
        b8 = jnp.pad(bn1_b[l], (0, 4))
        mx2 = _sp(g8 * (st[2] - mu) * rstd + b8)
        p = jnp.stack([mu, rstd, g8, b8, mx2,
                       jnp.zeros_like(mu), jnp.zeros_like(mu),
                       jnp.zeros_like(mu)], axis=0)

        e = pl.pallas_call(
            _pass2_body,
            grid=(grid_e,),
            in_specs=[_row_spec(_BM, 8), _full_spec((8, 8))],
            out_specs=_row_spec(_BM, 8),
            out_shape=jax.ShapeDtypeStruct((_E, 8), f32),
        )(ar, p)

        e4 = e[:, :_H]
        s = jax.ops.segment_sum(e4, idx_i, num_segments=_N)
        coef = e4 / (jnp.take(s, idx_i, axis=0) + 1e-16)
        coef8 = jnp.pad(coef, ((0, 0), (0, 4)))

        msg = pl.pallas_call(
            _pass3_body,
            grid=(grid_e,),
            in_specs=[qj_spec, _row_spec(_BM, _D),
                      _full_spec((_D, 256)), _row_spec(_BM, 8)],
            out_specs=_row_spec(_BM, _D),
            out_shape=jax.ShapeDtypeStruct((_E, _D), f32),
        )(qcat, out_e, w_bot, coef8)

        aggr = jax.ops.segment_sum(msg, idx_i, num_segments=_N)

        linp = jnp.pad(dgn_lin[l], ((0, 0), (0, 16 - _GROUPS)))
        grid_n = _N // _BM
        o640, hpre, st2 = pl.pallas_call(
            _dgn1_body,
            grid=(grid_n,),
            in_specs=[_row_spec(_BM, _D), _full_spec((1, _D)),
                      _full_spec((_D, 16))],
            out_specs=[_row_spec(_BM, _GROUPS * _D), _row_spec(_BM, _D),
                       _full_spec((8, _GROUPS * _D))],
            out_shape=[jax.ShapeDtypeStruct((_N, _GROUPS * _D), f32),
                       jax.ShapeDtypeStruct((_N, _D), f32),
                       jax.ShapeDtypeStruct((8, _GROUPS * _D), f32)],
        )(aggr, bias_stack[l].reshape(1, _D), linp)

        mu2 = st2[0] / _N
        var2 = st2[1] / _N - mu2 * mu2
        rstd2 = 1.0 / jnp.sqrt(var2 + 1e-5)
        p2 = jnp.stack([mu2, rstd2, dgn_g[l], dgn_b[l],
                        jnp.zeros_like(mu2), jnp.zeros_like(mu2),
                        jnp.zeros_like(mu2), jnp.zeros_like(mu2)], axis=0)

        out_x = pl.pallas_call(
            _dgn2_body,
            grid=(grid_n,),
            in_specs=[_row_spec(_BM, _GROUPS * _D), _row_spec(_BM, _D),
                      _row_spec(_BM, _D), _full_spec((8, _GROUPS * _D))],
            out_specs=_row_spec(_BM, _D),
            out_shape=jax.ShapeDtypeStruct((_N, _D), f32),
        )(o640, hpre, prev, p2)
        prev = out_x

    glp = jnp.pad(glbl_x, ((0, 0), (0, 112 - glbl_x.shape[1])))
    w0p = jnp.pad(ga_w0, ((0, 176 - ga_w0.shape[0]), (0, 0)))
    w2p = jnp.pad(ga_w2, ((0, 0), (0, 7)))
    b2p = jnp.pad(ga_b2, (0, 7)).reshape(1, 8)
    grid_n = _N // _BM
    a3, stf = pl.pallas_call(
        _gatt_body,
        grid=(grid_n,),
        in_specs=[_row_spec(_BM, _D), _row_spec(_BM, 112),
                  _full_spec((176, _D)), _full_spec((1, _D)),
                  _full_spec((_D, _D)), _full_spec((1, _D)),
                  _full_spec((_D, 8)), _full_spec((1, 8))],
        out_specs=[_row_spec(_BM, 8), _full_spec((8, 8))],
        out_shape=[jax.ShapeDtypeStruct((_N, 8), f32),
                   jax.ShapeDtypeStruct((8, 8), f32)],
    )(out_x, glp, w0p, ga_b0.reshape(1, _D), ga_w1, ga_b1.reshape(1, _D),
      w2p, b2p)

    mg = stf[0, 0]
    ea = jnp.exp(a3[:, 0] - mg)
    sg = jax.ops.segment_sum(ea, batch, num_segments=_G)
    coefg = (ea / (jnp.take(sg, batch) + 1e-16))[:, None]
    pooled = jax.ops.segment_sum(out_x * coefg, batch, num_segments=_G)

    pooled = jnp.pad(pooled, ((0, 4), (0, 0)))
    hf = _mm(pooled, post_w, post_b, True, bm=104)
    out_wp = jnp.pad(out_w, ((0, 0), (0, 7)))
    out_bp = jnp.pad(out_b, (0, 7))
    res = _mm(hf, out_wp, out_bp, False, bm=104)
    return res[:_G, 0]
